# baseline (speedup 1.0000x reference)
import functools

import jax
import jax.numpy as jnp
from jax import lax
from jax.experimental import pallas as pl
from jax.experimental.pallas import tpu as pltpu
from jax.experimental.pallas import tpu_sc as plsc

B = 16384
ED = 10
EDP = 16
RPS = 8
NC = 2
NS = 16
NW = NC * NS
BPW = B // NW
ICH = 128
NCH = BPW // ICH
RT = 512
BT = 2048


def _slab(v):
    return RT * (-(-v // (RPS * RT)))


def _repack_kernel(*refs):
    in_refs, e_ref, out_ref = refs[:RPS], refs[RPS], refs[RPS + 1]
    dot = functools.partial(
        lax.dot_general,
        dimension_numbers=(((1,), (0,)), ((), ())),
        preferred_element_type=jnp.float32,
        precision=lax.Precision.HIGHEST,
    )
    acc = dot(in_refs[0][...], e_ref[0])
    for j in range(1, RPS):
        acc += dot(in_refs[j][...], e_ref[j])
    out_ref[...] = acc


def _repack(table):
    v = table.shape[0]
    slab = _slab(v)
    nblk = slab // RT
    maxb = (v - 1) // RT
    e = jnp.zeros((RPS, ED, RPS * EDP), jnp.float32)
    rows = jnp.arange(ED)
    for j in range(RPS):
        e = e.at[j, rows, EDP * j + rows].set(1.0)

    in_spec = lambda j: pl.BlockSpec(
        (RT, ED), lambda i, j=j: (jnp.minimum(j * nblk + i, maxb), 0))
    return pl.pallas_call(
        _repack_kernel,
        grid=(nblk,),
        in_specs=[in_spec(j) for j in range(RPS)]
        + [pl.BlockSpec((RPS, ED, RPS * EDP), lambda i: (0, 0, 0))],
        out_specs=pl.BlockSpec((RT, RPS * EDP), lambda i: (i, 0)),
        out_shape=jax.ShapeDtypeStruct((slab, RPS * EDP), jnp.float32),
    )(*([table] * RPS), e)


def _sc_gather5(tables, superidx):
    mesh = plsc.VectorSubcoreMesh(core_axis_name="c", subcore_axis_name="s")

    @functools.partial(
        pl.kernel,
        mesh=mesh,
        out_type=[jax.ShapeDtypeStruct((B, RPS * EDP), jnp.float32)] * 5,
        scratch_types=(
            [pltpu.VMEM((NCH, ICH), jnp.int32) for _ in range(5)]
            + [pltpu.VMEM((ICH, RPS * EDP), jnp.float32) for _ in range(2)]
            + [pltpu.SemaphoreType.DMA for _ in range(2)]
        ),
    )
    def gather5(t0, t1, t2, t3, t4, i0, i1, i2, i3, i4,
                o0, o1, o2, o3, o4,
                iv0, iv1, iv2, iv3, iv4, b0, b1, s0, s1):
        tabs = (t0, t1, t2, t3, t4)
        idxs = (i0, i1, i2, i3, i4)
        outs = (o0, o1, o2, o3, o4)
        ivs = (iv0, iv1, iv2, iv3, iv4)
        bufs = (b0, b1)
        sems = (s0, s1)
        wid = lax.axis_index("s") * NC + lax.axis_index("c")
        for t in range(5):
            pltpu.sync_copy(idxs[t].at[pl.ds(wid * NCH, NCH)], ivs[t])
        chunks = [(t, j) for t in range(5) for j in range(NCH)]
        n = len(chunks)
        copies = [None, None]

        def fire(k):
            t, j = chunks[k]
            s = k % 2
            copies[s] = pltpu.async_copy(
                tabs[t].at[ivs[t].at[j]], bufs[s], sems[s])

        fire(0)
        for k in range(1, n + 1):
            if k < n:
                fire(k)
            s = (k - 1) % 2
            copies[s].wait()
            t, j = chunks[k - 1]
            pltpu.sync_copy(
                bufs[s], outs[t].at[pl.ds(wid * BPW + j * ICH, ICH)])

    return gather5(*tables, *superidx)


def _mlp_kernel(u_ref, tid_ref, tcate_ref, sid_ref, scate_ref,
                ur_ref, tidr_ref, tcater_ref, sidr_ref, scater_ref,
                wsu_ref, wss1_ref, wss2_ref, wst1_ref, wst2_ref,
                wtu_ref, wtt1_ref, wtt2_ref, wts1_ref, wts2_ref,
                ws1_ref, h1_ref, wt1_ref,
                ws2_ref, h2_ref, wt2_ref,
                ws3_ref, h3_ref, wt3_ref,
                spw_ref, spb_ref, tpw_ref, tpb_ref,
                rs_ref, rt_ref):
    dot = functools.partial(
        lax.dot_general,
        dimension_numbers=(((1,), (0,)), ((), ())),
        preferred_element_type=jnp.float32,
        precision=lax.Precision.HIGHEST,
    )

    def select(g_ref, r_ref):
        g = g_ref[...]
        r = r_ref[...]
        acc = jnp.zeros((BT, EDP), jnp.float32)
        for o in range(RPS):
            acc += jnp.where(r == o, g[:, o * EDP:(o + 1) * EDP], 0.0)
        return acc

    u = select(u_ref, ur_ref)
    tid = select(tid_ref, tidr_ref)
    tcate = select(tcate_ref, tcater_ref)
    sid = select(sid_ref, sidr_ref)
    scate = select(scate_ref, scater_ref)
    xs = jax.nn.relu(
        dot(u, wsu_ref[...]) + dot(sid, wss1_ref[...]) + dot(scate, wss2_ref[...])
        + dot(tid, wst1_ref[...]) + dot(tcate, wst2_ref[...]))
    xt = jax.nn.relu(
        dot(u, wtu_ref[...]) + dot(tid, wtt1_ref[...]) + dot(tcate, wtt2_ref[...])
        + dot(sid, wts1_ref[...]) + dot(scate, wts2_ref[...]))
    for ws_r, h_r, wt_r in ((ws1_ref, h1_ref, wt1_ref),
                            (ws2_ref, h2_ref, wt2_ref),
                            (ws3_ref, h3_ref, wt3_ref)):
        xs, xt = (jax.nn.relu(dot(xs, ws_r[...]) + dot(xt, h_r[...])),
                  jax.nn.relu(dot(xt, wt_r[...]) + dot(xs, h_r[...])))
    rs_ref[...] = jax.nn.sigmoid(dot(xs, spw_ref[...]) + spb_ref[...])
    rt_ref[...] = jax.nn.sigmoid(dot(xt, tpw_ref[...]) + tpb_ref[...])


def kernel(userid, t_can_id, t_can_cate, s_can_id, s_can_cate,
           user_emb, t_itemid_emb, t_itemcate_emb, s_itemid_emb, s_itemcate_emb,
           ws0, h0, wt0, ws1, h1, wt1, ws2, h2, wt2, ws3, h3, wt3,
           s_pred_w, s_pred_b, t_pred_w, t_pred_b):
    tabs = (user_emb, t_itemid_emb, t_itemcate_emb, s_itemid_emb, s_itemcate_emb)
    idxs = (userid, t_can_id, t_can_cate, s_can_id, s_can_cate)
    slabs = [_slab(t.shape[0]) for t in tabs]
    sups = tuple((i % s).reshape(B // ICH, ICH) for i, s in zip(idxs, slabs))
    rems = tuple((i // s).reshape(B, 1) for i, s in zip(idxs, slabs))
    g_u, g_tid, g_tcate, g_sid, g_scate = _sc_gather5(
        tuple(_repack(t) for t in tabs), sups)

    pad_w = lambda w: jnp.pad(w, ((0, EDP - ED), (0, 0)))
    wsu = pad_w((ws0[:, :ED] + h0[:, :ED]).T)
    wss1 = pad_w(ws0[:, ED:2 * ED].T)
    wss2 = pad_w(ws0[:, 2 * ED:3 * ED].T)
    wst1 = pad_w(h0[:, ED:2 * ED].T)
    wst2 = pad_w(h0[:, 2 * ED:3 * ED].T)
    wtu = pad_w((wt0[:, :ED] + h0[:, :ED]).T)
    wtt1 = pad_w(wt0[:, ED:2 * ED].T)
    wtt2 = pad_w(wt0[:, 2 * ED:3 * ED].T)
    wts1 = pad_w(h0[:, ED:2 * ED].T)
    wts2 = pad_w(h0[:, 2 * ED:3 * ED].T)

    bspec = pl.BlockSpec((BT, RPS * EDP), lambda i: (i, 0))
    rspec = pl.BlockSpec((BT, 1), lambda i: (i, 0))
    wspec = lambda a: pl.BlockSpec(a.shape, lambda i: (0,) * a.ndim)
    warr = (wsu, wss1, wss2, wst1, wst2, wtu, wtt1, wtt2, wts1, wts2,
            ws1.T, h1.T, wt1.T, ws2.T, h2.T, wt2.T, ws3.T, h3.T, wt3.T,
            s_pred_w.T, s_pred_b, t_pred_w.T, t_pred_b)
    rs, rt = pl.pallas_call(
        _mlp_kernel,
        grid=(B // BT,),
        in_specs=([bspec] * 5 + [rspec] * 5 + [wspec(a) for a in warr]),
        out_specs=[rspec] * 2,
        out_shape=[jax.ShapeDtypeStruct((B, 1), jnp.float32)] * 2,
    )(g_u, g_tid, g_tcate, g_sid, g_scate, *rems, *warr)
    return rs.reshape(B), rt.reshape(B)


# speedup vs baseline: 1.2792x; 1.2792x over previous
import dataclasses
import functools

import jax
import jax.numpy as jnp
from jax import lax
from jax.experimental import pallas as pl
from jax.experimental.pallas import tpu as pltpu
from jax.experimental.pallas import tpu_sc as plsc

B = 16384
ED = 10
EDP = 10
NC = 2
NS = 16
NW = NC * NS
BPW = B // NW
GRP = 32
BT = 2048


def _sc_gather5(tables, indices):
    mesh = plsc.VectorSubcoreMesh(core_axis_name="c", subcore_axis_name="s")
    cp = pltpu.CompilerParams()
    if "needs_layout_passes" in pltpu.CompilerParams.__dataclass_fields__:
        cp = dataclasses.replace(cp, needs_layout_passes=False)

    @functools.partial(
        pl.kernel,
        mesh=mesh,
        out_type=[jax.ShapeDtypeStruct((B, ED), jnp.float32)] * 5,
        scratch_types=[
            pltpu.VMEM((BPW,), jnp.int32),
            pltpu.SemaphoreType.DMA,
        ],
        compiler_params=cp,
    )
    def gather5(t0, t1, t2, t3, t4, i0, i1, i2, i3, i4,
                o0, o1, o2, o3, o4, idx_v, sem):
        tabs = (t0, t1, t2, t3, t4)
        idxs = (i0, i1, i2, i3, i4)
        outs = (o0, o1, o2, o3, o4)
        wid = lax.axis_index("s") * NC + lax.axis_index("c")
        base = wid * BPW
        lane = lax.iota(jnp.int32, 16)
        for t in range(5):
            pltpu.sync_copy(idxs[t].at[pl.ds(base, BPW)], idx_v)

            @pl.loop(0, BPW, step=GRP)
            def _(r0, tab=tabs[t], out=outs[t]):
                copies = []
                for g in range(GRP // 16):
                    v = idx_v[pl.ds(r0 + g * 16, 16)]
                    for k in range(16):
                        ik = jnp.sum(jnp.where(lane == k, v, 0))
                        copies.append(pltpu.async_copy(
                            tab.at[pl.ds(ik, 1)],
                            out.at[pl.ds(base + r0 + g * 16 + k, 1)], sem))
                for c in copies:
                    c.wait()

    return gather5(*tables, *indices)


def _mlp_kernel(u_ref, tid_ref, tcate_ref, sid_ref, scate_ref,
                wsu_ref, wss1_ref, wss2_ref, wst1_ref, wst2_ref,
                wtu_ref, wtt1_ref, wtt2_ref, wts1_ref, wts2_ref,
                ws1_ref, h1_ref, wt1_ref,
                ws2_ref, h2_ref, wt2_ref,
                ws3_ref, h3_ref, wt3_ref,
                spw_ref, spb_ref, tpw_ref, tpb_ref,
                rs_ref, rt_ref):
    dot = functools.partial(
        lax.dot_general,
        dimension_numbers=(((1,), (0,)), ((), ())),
        preferred_element_type=jnp.float32,
        precision=lax.Precision.HIGHEST,
    )
    u = u_ref[...]
    tid = tid_ref[...]
    tcate = tcate_ref[...]
    sid = sid_ref[...]
    scate = scate_ref[...]
    xs = jax.nn.relu(
        dot(u, wsu_ref[...]) + dot(sid, wss1_ref[...]) + dot(scate, wss2_ref[...])
        + dot(tid, wst1_ref[...]) + dot(tcate, wst2_ref[...]))
    xt = jax.nn.relu(
        dot(u, wtu_ref[...]) + dot(tid, wtt1_ref[...]) + dot(tcate, wtt2_ref[...])
        + dot(sid, wts1_ref[...]) + dot(scate, wts2_ref[...]))
    for ws_r, h_r, wt_r in ((ws1_ref, h1_ref, wt1_ref),
                            (ws2_ref, h2_ref, wt2_ref),
                            (ws3_ref, h3_ref, wt3_ref)):
        xs, xt = (jax.nn.relu(dot(xs, ws_r[...]) + dot(xt, h_r[...])),
                  jax.nn.relu(dot(xt, wt_r[...]) + dot(xs, h_r[...])))
    rs_ref[...] = jax.nn.sigmoid(dot(xs, spw_ref[...]) + spb_ref[...])
    rt_ref[...] = jax.nn.sigmoid(dot(xt, tpw_ref[...]) + tpb_ref[...])


def kernel(userid, t_can_id, t_can_cate, s_can_id, s_can_cate,
           user_emb, t_itemid_emb, t_itemcate_emb, s_itemid_emb, s_itemcate_emb,
           ws0, h0, wt0, ws1, h1, wt1, ws2, h2, wt2, ws3, h3, wt3,
           s_pred_w, s_pred_b, t_pred_w, t_pred_b):
    u, tid, tcate, sid, scate = _sc_gather5(
        (user_emb, t_itemid_emb, t_itemcate_emb, s_itemid_emb, s_itemcate_emb),
        (userid, t_can_id, t_can_cate, s_can_id, s_can_cate))

    wsu = (ws0[:, :ED] + h0[:, :ED]).T
    wss1 = ws0[:, ED:2 * ED].T
    wss2 = ws0[:, 2 * ED:3 * ED].T
    wst1 = h0[:, ED:2 * ED].T
    wst2 = h0[:, 2 * ED:3 * ED].T
    wtu = (wt0[:, :ED] + h0[:, :ED]).T
    wtt1 = wt0[:, ED:2 * ED].T
    wtt2 = wt0[:, 2 * ED:3 * ED].T
    wts1 = h0[:, ED:2 * ED].T
    wts2 = h0[:, 2 * ED:3 * ED].T

    bspec = pl.BlockSpec((BT, ED), lambda i: (i, 0))
    rspec = pl.BlockSpec((BT, 1), lambda i: (i, 0))
    wspec = lambda a: pl.BlockSpec(a.shape, lambda i: (0,) * a.ndim)
    warr = (wsu, wss1, wss2, wst1, wst2, wtu, wtt1, wtt2, wts1, wts2,
            ws1.T, h1.T, wt1.T, ws2.T, h2.T, wt2.T, ws3.T, h3.T, wt3.T,
            s_pred_w.T, s_pred_b, t_pred_w.T, t_pred_b)
    rs, rt = pl.pallas_call(
        _mlp_kernel,
        grid=(B // BT,),
        in_specs=([bspec] * 5 + [wspec(a) for a in warr]),
        out_specs=[rspec] * 2,
        out_shape=[jax.ShapeDtypeStruct((B, 1), jnp.float32)] * 2,
    )(u, tid, tcate, sid, scate, *warr)
    return rs.reshape(B), rt.reshape(B)


# speedup vs baseline: 1.3633x; 1.0658x over previous
import dataclasses
import functools

import jax
import jax.numpy as jnp
from jax import lax
from jax.experimental import pallas as pl
from jax.experimental.pallas import tpu as pltpu
from jax.experimental.pallas import tpu_sc as plsc

B = 16384
ED = 10
EDP = 10
NC = 2
NS = 16
NW = NC * NS
BPW = B // NW
GRP = 32
BT = 2048


def _sc_gather5(tables, indices):
    mesh = plsc.VectorSubcoreMesh(core_axis_name="c", subcore_axis_name="s")
    cp = pltpu.CompilerParams()
    if "needs_layout_passes" in pltpu.CompilerParams.__dataclass_fields__:
        cp = dataclasses.replace(cp, needs_layout_passes=False)

    @functools.partial(
        pl.kernel,
        mesh=mesh,
        out_type=[jax.ShapeDtypeStruct((B, ED), jnp.float32)] * 5,
        scratch_types=[
            pltpu.VMEM((BPW,), jnp.int32),
            pltpu.SemaphoreType.DMA,
        ],
        compiler_params=cp,
    )
    def gather5(t0, t1, t2, t3, t4, i0, i1, i2, i3, i4,
                o0, o1, o2, o3, o4, idx_v, sem):
        tabs = (t0, t1, t2, t3, t4)
        idxs = (i0, i1, i2, i3, i4)
        outs = (o0, o1, o2, o3, o4)
        wid = lax.axis_index("s") * NC + lax.axis_index("c")
        base = wid * BPW
        lane = lax.iota(jnp.int32, 16)
        for t in range(5):
            pltpu.sync_copy(idxs[t].at[pl.ds(base, BPW)], idx_v)

            @pl.loop(0, BPW, step=GRP)
            def _(r0, tab=tabs[t], out=outs[t]):
                copies = []
                for g in range(GRP // 16):
                    v = idx_v[pl.ds(r0 + g * 16, 16)]
                    for k in range(16):
                        ik = jnp.sum(jnp.where(lane == k, v, 0))
                        copies.append(pltpu.async_copy(
                            tab.at[pl.ds(ik, 1)],
                            out.at[pl.ds(base + r0 + g * 16 + k, 1)], sem))
                for c in copies:
                    c.wait()

    return gather5(*tables, *indices)


def _mlp_kernel(u_ref, tid_ref, tcate_ref, sid_ref, scate_ref,
                wsu_ref, wss1_ref, wss2_ref, wst1_ref, wst2_ref,
                wtu_ref, wtt1_ref, wtt2_ref, wts1_ref, wts2_ref,
                ws1_ref, h1_ref, wt1_ref,
                ws2_ref, h2_ref, wt2_ref,
                ws3_ref, h3_ref, wt3_ref,
                spw_ref, spb_ref, tpw_ref, tpb_ref,
                rs_ref, rt_ref):
    dot = functools.partial(
        lax.dot_general,
        dimension_numbers=(((1,), (0,)), ((), ())),
        preferred_element_type=jnp.float32,
    )
    u = u_ref[...]
    tid = tid_ref[...]
    tcate = tcate_ref[...]
    sid = sid_ref[...]
    scate = scate_ref[...]
    xs = jax.nn.relu(
        dot(u, wsu_ref[...]) + dot(sid, wss1_ref[...]) + dot(scate, wss2_ref[...])
        + dot(tid, wst1_ref[...]) + dot(tcate, wst2_ref[...]))
    xt = jax.nn.relu(
        dot(u, wtu_ref[...]) + dot(tid, wtt1_ref[...]) + dot(tcate, wtt2_ref[...])
        + dot(sid, wts1_ref[...]) + dot(scate, wts2_ref[...]))
    for ws_r, h_r, wt_r in ((ws1_ref, h1_ref, wt1_ref),
                            (ws2_ref, h2_ref, wt2_ref),
                            (ws3_ref, h3_ref, wt3_ref)):
        xs, xt = (jax.nn.relu(dot(xs, ws_r[...]) + dot(xt, h_r[...])),
                  jax.nn.relu(dot(xt, wt_r[...]) + dot(xs, h_r[...])))
    rs_ref[...] = jax.nn.sigmoid(dot(xs, spw_ref[...]) + spb_ref[...])
    rt_ref[...] = jax.nn.sigmoid(dot(xt, tpw_ref[...]) + tpb_ref[...])


def kernel(userid, t_can_id, t_can_cate, s_can_id, s_can_cate,
           user_emb, t_itemid_emb, t_itemcate_emb, s_itemid_emb, s_itemcate_emb,
           ws0, h0, wt0, ws1, h1, wt1, ws2, h2, wt2, ws3, h3, wt3,
           s_pred_w, s_pred_b, t_pred_w, t_pred_b):
    u, tid, tcate, sid, scate = _sc_gather5(
        (user_emb, t_itemid_emb, t_itemcate_emb, s_itemid_emb, s_itemcate_emb),
        (userid, t_can_id, t_can_cate, s_can_id, s_can_cate))

    wsu = (ws0[:, :ED] + h0[:, :ED]).T
    wss1 = ws0[:, ED:2 * ED].T
    wss2 = ws0[:, 2 * ED:3 * ED].T
    wst1 = h0[:, ED:2 * ED].T
    wst2 = h0[:, 2 * ED:3 * ED].T
    wtu = (wt0[:, :ED] + h0[:, :ED]).T
    wtt1 = wt0[:, ED:2 * ED].T
    wtt2 = wt0[:, 2 * ED:3 * ED].T
    wts1 = h0[:, ED:2 * ED].T
    wts2 = h0[:, 2 * ED:3 * ED].T

    bspec = pl.BlockSpec((BT, ED), lambda i: (i, 0))
    rspec = pl.BlockSpec((BT, 1), lambda i: (i, 0))
    wspec = lambda a: pl.BlockSpec(a.shape, lambda i: (0,) * a.ndim)
    warr = (wsu, wss1, wss2, wst1, wst2, wtu, wtt1, wtt2, wts1, wts2,
            ws1.T, h1.T, wt1.T, ws2.T, h2.T, wt2.T, ws3.T, h3.T, wt3.T,
            s_pred_w.T, s_pred_b, t_pred_w.T, t_pred_b)
    rs, rt = pl.pallas_call(
        _mlp_kernel,
        grid=(B // BT,),
        in_specs=([bspec] * 5 + [wspec(a) for a in warr]),
        out_specs=[rspec] * 2,
        out_shape=[jax.ShapeDtypeStruct((B, 1), jnp.float32)] * 2,
    )(u, tid, tcate, sid, scate, *warr)
    return rs.reshape(B), rt.reshape(B)
